# baseline (speedup 1.0000x reference)
import math
import functools

import jax
import jax.numpy as jnp
from jax.experimental import pallas as pl
from jax.experimental.pallas import tpu as pltpu
from jax.experimental.pallas import tpu_sc as plsc

B, L = 1, 2048
D_MODEL, D_KEY, D_QUERY, N_HEADS = 768, 64, 64, 8
RANK, RFF_DIM, K_MAX = 32, 64, 8
LSH_BUCKETS, LSH_BANDWIDTH, LSH_KEY_DIM, N_LSH_HASHES = 8, 16.0, 64, 2
NB = LSH_BUCKETS ** N_LSH_HASHES
NC = NB * K_MAX
RFF_SCALE = math.sqrt(2.0 / RFF_DIM)
SIM_SCALE = math.sqrt(RFF_DIM)
CHUNK = 256
N_WORKERS = 32
GB = (N_HEADS * NC) // N_WORKERS


def _hash_digits(x, lshv):
    xb = (jax.lax.bitcast_convert_type(x, jnp.int32) > 0).astype(jnp.float32)
    proj = jnp.dot(xb, lshv, preferred_element_type=jnp.float32)
    return jnp.floor(proj / LSH_BANDWIDTH) % LSH_BUCKETS


def _combine(width, scale=1.0):
    r = jax.lax.broadcasted_iota(jnp.int32, (N_LSH_HASHES, width), 0)
    return jnp.where(r == 0, scale * LSH_BUCKETS, scale).astype(jnp.float32)


def _key_kernel(xk_ref, xv_ref, wk_ref, bk_ref, wv_ref, bv_ref, omega_ref,
                rffb_ref, lsh_ref, krv_ref, idx_ref, cnt_ref):
    hid = pl.program_id(0)
    kh = jnp.dot(xk_ref[...], wk_ref[0],
                 preferred_element_type=jnp.float32) + bk_ref[0]
    vh = jnp.dot(xv_ref[...], wv_ref[0],
                 preferred_element_type=jnp.float32) + bv_ref[0]
    lshv = lsh_ref[0]

    hk = _hash_digits(kh, lshv)
    ck_bc64 = jnp.dot(hk, _combine(NB),
                      preferred_element_type=jnp.float32)
    ck8_bc = jnp.dot(hk, _combine(NC, scale=float(K_MAX)),
                     preferred_element_type=jnp.float32)

    iota64 = jax.lax.broadcasted_iota(jnp.int32, (L, NB), 1)
    onehot = (ck_bc64.astype(jnp.int32) == iota64).astype(jnp.float32)
    tril = (jax.lax.broadcasted_iota(jnp.int32, (CHUNK, CHUNK), 0)
            >= jax.lax.broadcasted_iota(jnp.int32, (CHUNK, CHUNK), 1)
            ).astype(jnp.float32)
    counts = jnp.zeros((1, NB), jnp.float32)
    ranks = []
    for c in range(L // CHUNK):
        blk = onehot[c * CHUNK:(c + 1) * CHUNK]
        csum = jnp.dot(tril, blk, preferred_element_type=jnp.float32) + counts
        ranks.append(jnp.sum(csum * blk, axis=1, keepdims=True))
        counts = counts + jnp.sum(blk, axis=0, keepdims=True)
    rank = jnp.concatenate(ranks, axis=0)

    slotfix = jnp.where(rank <= K_MAX, rank - 1.0, 4.0 * NC)
    ones_row = jnp.full((1, NC), 1.0, jnp.float32)
    slot_bc = ck8_bc + jnp.dot(slotfix, ones_row,
                               preferred_element_type=jnp.float32)
    col = jax.lax.broadcasted_iota(jnp.int32, (L, NC), 1)
    sel = (slot_bc.astype(jnp.int32) == col).astype(jnp.float32)
    j_row = jax.lax.broadcasted_iota(jnp.int32, (1, L), 1).astype(jnp.float32)
    key_of_slot = jnp.dot(j_row, sel,
                          preferred_element_type=jnp.float32)

    kr = jnp.cos(jnp.dot(kh, omega_ref[0],
                         preferred_element_type=jnp.float32)
                 + rffb_ref[0])
    krv_ref[0] = jnp.concatenate([kr, vh], axis=1)
    idx_ref[0] = key_of_slot.astype(jnp.int32) + hid * L
    cnt_ref[0] = counts


def _query_kernel(xq_ref, wq_ref, bq_ref, omega_ref, rffb_ref, lsh_ref,
                  qr_ref, hq_ref):
    qh = jnp.dot(xq_ref[...], wq_ref[0],
                 preferred_element_type=jnp.float32) + bq_ref[0]
    qr_ref[0] = jnp.cos(jnp.dot(qh, omega_ref[0],
                                preferred_element_type=jnp.float32)
                        + rffb_ref[0]) * (RFF_SCALE * RFF_SCALE / SIM_SCALE)
    hq_ref[0] = _hash_digits(qh, lsh_ref[0])


def _sc_gather(krv_all, idx):
    mesh = plsc.VectorSubcoreMesh(core_axis_name="c", subcore_axis_name="s")
    n_rows = N_HEADS * NC

    @functools.partial(
        pl.kernel, mesh=mesh,
        out_type=jax.ShapeDtypeStruct((n_rows, 2 * D_KEY), jnp.float32),
        scratch_types=[pltpu.VMEM((GB,), jnp.int32),
                       pltpu.VMEM((GB, 2 * D_KEY), jnp.float32),
                       pltpu.SemaphoreType.DMA],
    )
    def k(krv_hbm, idx_hbm, o_hbm, idx_v, rows_v, sem):
        wid = jax.lax.axis_index("s") * 2 + jax.lax.axis_index("c")
        base = wid * GB
        pltpu.sync_copy(idx_hbm.at[pl.ds(base, GB)], idx_v)
        pltpu.async_copy(krv_hbm.at[idx_v], rows_v, sem).wait()
        pltpu.sync_copy(rows_v, o_hbm.at[pl.ds(base, GB)])

    return k(krv_all, idx)


def _attn_kernel(qr_ref, hq_ref, krvt_ref, cnt_ref, uv_ref,
                 vv_ref, wo_ref, bo_ref, o_ref):
    hid = pl.program_id(0)
    qr = qr_ref[0]
    krvt = krvt_ref[0]
    krt = krvt[:, :D_KEY]
    cvt = krvt[:, D_KEY:]
    counts = cnt_ref[0]

    rowb = (jax.lax.broadcasted_iota(jnp.int32, (NC, NB), 0) >> 3)
    expand_r = (rowb == jax.lax.broadcasted_iota(jnp.int32, (NC, NB), 1)
                ).astype(jnp.float32)
    cnt_rows = jax.lax.dot_general(expand_r, counts,
                                   (((1,), (1,)), ((), ())),
                                   preferred_element_type=jnp.float32)
    slot_rows = (jax.lax.broadcasted_iota(jnp.int32, (NC, 1), 0)
                 % K_MAX).astype(jnp.float32)
    occ_rows = cnt_rows > slot_rows
    cv_safe = jnp.where(occ_rows, cvt, 0.0)

    col_row = jax.lax.broadcasted_iota(jnp.int32, (1, NC), 1)
    expand_c = (jax.lax.broadcasted_iota(jnp.int32, (NB, NC), 0)
                == jax.lax.broadcasted_iota(jnp.int32, (NB, NC), 1) // K_MAX
                ).astype(jnp.float32)
    cnt_col = jnp.dot(counts, expand_c, preferred_element_type=jnp.float32)
    ctab = jnp.where(cnt_col > (col_row % K_MAX).astype(jnp.float32),
                     col_row >> 3, -1)

    cq_bc = jnp.dot(hq_ref[0], _combine(NC),
                    preferred_element_type=jnp.float32)
    sim = jax.lax.dot_general(qr, krt, (((1,), (1,)), ((), ())),
                              preferred_element_type=jnp.float32)
    simm = jnp.where(cq_bc.astype(jnp.int32) == ctab, sim, -jnp.inf)
    m = jnp.max(simm, axis=1, keepdims=True)
    p = jnp.exp(simm - m)
    wvu = jnp.dot(p, cv_safe, preferred_element_type=jnp.float32)
    wv = wvu * (1.0 / jnp.sum(p, axis=1, keepdims=True))
    r = jnp.dot(wv, uv_ref[0], preferred_element_type=jnp.float32)

    n = jnp.dot(vv_ref[0], wo_ref[...], preferred_element_type=jnp.float32)
    acc = jnp.dot(r, n, preferred_element_type=jnp.float32)

    @pl.when(hid == 0)
    def _():
        o_ref[...] = acc + bo_ref[...]

    @pl.when(hid != 0)
    def _():
        o_ref[...] += acc


def _head_spec(d2, d3):
    return pl.BlockSpec((1, d2, d3), lambda h: (h, 0, 0))


@functools.partial(jax.jit, static_argnames=("interpret",))
def kernel(query, key, value, Wq, bq, Wk, bk, Wv, bv, Uv, Vv, omega,
           rff_bias, lsh_vecs, Wo, bo, interpret=False):
    x_q = query.reshape(L, D_MODEL)
    x_k = key.reshape(L, D_MODEL)
    x_v = value.reshape(L, D_MODEL)
    wq_t = Wq.reshape(D_MODEL, N_HEADS, D_QUERY).transpose(1, 0, 2)
    wk_t = Wk.reshape(D_MODEL, N_HEADS, D_KEY).transpose(1, 0, 2)
    wv_t = Wv.reshape(D_MODEL, N_HEADS, D_KEY).transpose(1, 0, 2)
    bq3 = bq.reshape(N_HEADS, 1, D_QUERY)
    bk3 = bk.reshape(N_HEADS, 1, D_KEY)
    bv3 = bv.reshape(N_HEADS, 1, D_KEY)
    bo2 = bo.reshape(1, -1)
    rffb3 = rff_bias.reshape(N_HEADS, 1, RFF_DIM)

    krv_all, idx, cnts = pl.pallas_call(
        _key_kernel,
        grid=(N_HEADS,),
        in_specs=[
            pl.BlockSpec((L, D_MODEL), lambda h: (0, 0)),
            pl.BlockSpec((L, D_MODEL), lambda h: (0, 0)),
            _head_spec(D_MODEL, D_KEY),
            _head_spec(1, D_KEY),
            _head_spec(D_MODEL, D_KEY),
            _head_spec(1, D_KEY),
            _head_spec(D_KEY, RFF_DIM),
            _head_spec(1, RFF_DIM),
            _head_spec(LSH_KEY_DIM, N_LSH_HASHES),
        ],
        out_specs=[
            pl.BlockSpec((1, L, 2 * D_KEY), lambda h: (h, 0, 0)),
            pl.BlockSpec((1, 1, NC), lambda h: (h, 0, 0)),
            pl.BlockSpec((1, 1, NB), lambda h: (h, 0, 0)),
        ],
        out_shape=[
            jax.ShapeDtypeStruct((N_HEADS, L, 2 * D_KEY), jnp.float32),
            jax.ShapeDtypeStruct((N_HEADS, 1, NC), jnp.int32),
            jax.ShapeDtypeStruct((N_HEADS, 1, NB), jnp.float32),
        ],
        compiler_params=pltpu.CompilerParams(
            fuse_transposed_lhs_in_matmul=True),
        interpret=interpret,
    )(x_k, x_v, wk_t, bk3, wv_t, bv3, omega, rffb3, lsh_vecs)

    qr, hq = pl.pallas_call(
        _query_kernel,
        grid=(N_HEADS,),
        in_specs=[
            pl.BlockSpec((L, D_MODEL), lambda h: (0, 0)),
            _head_spec(D_MODEL, D_QUERY),
            _head_spec(1, D_QUERY),
            _head_spec(D_KEY, RFF_DIM),
            _head_spec(1, RFF_DIM),
            _head_spec(LSH_KEY_DIM, N_LSH_HASHES),
        ],
        out_specs=[
            pl.BlockSpec((1, L, D_KEY), lambda h: (h, 0, 0)),
            pl.BlockSpec((1, L, N_LSH_HASHES), lambda h: (h, 0, 0)),
        ],
        out_shape=[
            jax.ShapeDtypeStruct((N_HEADS, L, D_KEY), jnp.float32),
            jax.ShapeDtypeStruct((N_HEADS, L, N_LSH_HASHES), jnp.float32),
        ],
        interpret=interpret,
    )(x_q, wq_t, bq3, omega, rffb3, lsh_vecs)

    krvt = _sc_gather(krv_all.reshape(N_HEADS * L, 2 * D_KEY),
                      idx.reshape(N_HEADS * NC))
    krvt = krvt.reshape(N_HEADS, NC, 2 * D_KEY)

    out = pl.pallas_call(
        _attn_kernel,
        grid=(N_HEADS,),
        in_specs=[
            pl.BlockSpec((1, L, D_KEY), lambda h: (h, 0, 0)),
            pl.BlockSpec((1, L, N_LSH_HASHES), lambda h: (h, 0, 0)),
            pl.BlockSpec((1, NC, 2 * D_KEY), lambda h: (h, 0, 0)),
            _head_spec(1, NB),
            _head_spec(D_KEY, RANK),
            _head_spec(RANK, D_MODEL),
            pl.BlockSpec((D_MODEL, D_MODEL), lambda h: (h, 0)),
            pl.BlockSpec((1, D_MODEL), lambda h: (0, 0)),
        ],
        out_specs=pl.BlockSpec((L, D_MODEL), lambda h: (0, 0)),
        out_shape=jax.ShapeDtypeStruct((L, D_MODEL), jnp.float32),
        interpret=interpret,
    )(qr, hq, krvt, cnts, Uv, Vv, Wo, bo2)

    return out.reshape(B, L, D_MODEL)
